# baseline (speedup 1.0000x reference)
import jax
import jax.numpy as jnp
from jax import lax
from jax.experimental import pallas as pl
from jax.experimental.pallas import tpu as pltpu
from jax.experimental.pallas import tpu_sc as plsc

N_NODES = 10000
IN_DIM = 256
HID_DIM = 128
OUT_DIM = 64
N_EDGES = 160000

NC = 2
NS = 16
EDGE_BLK = 128
NBUF = 3
EDGES_PER_TILE = -(-N_EDGES // NS)
IBLOCKS = -(-EDGES_PER_TILE // EDGE_BLK)
IBLOCKS = -(-IBLOCKS // NBUF) * NBUF
EDGES_PAD = NS * IBLOCKS * EDGE_BLK
NGROUPS = IBLOCKS // NBUF
ACC_ROWS = 10240
ROWS_PER_TILE = ACC_ROWS // NS
ZCHUNK = 128
CNT_W = 16
H1 = HID_DIM // 2
H1A = H1 + CNT_W
H2 = OUT_DIM // 2

_mesh = plsc.VectorSubcoreMesh(core_axis_name="c", subcore_axis_name="s",
                               num_cores=NC, num_subcores=NS)


def _seg_body(width, finish, y_hbm, src_hbm, dst_hbm, zacc_hbm, *rest):
  if finish:
    (hr_hbm, rcp_hbm, out_hbm, src_v, dst_v, gb0, gb1, gb2, rbuf, acc,
     gsem0, gsem1, gsem2, ssem0, ssem1, ssem2) = rest
  else:
    (out_hbm, src_v, dst_v, gb0, gb1, gb2, acc,
     gsem0, gsem1, gsem2, ssem0, ssem1, ssem2) = rest
  gbuf = (gb0, gb1, gb2)
  gsem = (gsem0, gsem1, gsem2)
  ssem = (ssem0, ssem1, ssem2)
  c = lax.axis_index("c")
  s = lax.axis_index("s")
  base = s * ROWS_PER_TILE

  pltpu.sync_copy(src_hbm.at[c, s], src_v)
  pltpu.sync_copy(dst_hbm.at[s], dst_v)
  for k in range(ROWS_PER_TILE // ZCHUNK):
    pltpu.sync_copy(zacc_hbm, acc.at[pl.ds(base + k * ZCHUNK, ZCHUNK)])
  plsc.subcore_barrier()

  for b in range(NBUF):
    pltpu.async_copy(y_hbm.at[src_v.at[b]], gbuf[b], gsem[b])

  def group(g, _):
    for b in range(NBUF):
      j = g * NBUF + b
      pltpu.make_async_copy(y_hbm.at[src_v.at[j]], gbuf[b], gsem[b]).wait()
      pltpu.async_copy(gbuf[b], acc.at[dst_v.at[j]], ssem[b], add=True)
    for b in range(NBUF):
      jn = (g + 1) * NBUF + b

      @pl.when(jn < IBLOCKS)
      def _():
        pltpu.make_async_copy(gbuf[b], acc.at[dst_v.at[jn]], ssem[b]).wait()
        pltpu.async_copy(y_hbm.at[src_v.at[jn]], gbuf[b], gsem[b])
    return 0

  lax.fori_loop(0, NGROUPS, group, 0)

  for b in range(NBUF):
    pltpu.make_async_copy(gbuf[b], acc.at[dst_v.at[0]], ssem[b]).wait()
  plsc.subcore_barrier()

  if not finish:
    pltpu.sync_copy(acc.at[pl.ds(base, ROWS_PER_TILE)],
                    out_hbm.at[c, pl.ds(base, ROWS_PER_TILE)])
    return

  for k in range(ROWS_PER_TILE // ZCHUNK):
    r0 = base + k * ZCHUNK
    pltpu.sync_copy(acc.at[pl.ds(r0, ZCHUNK)], gb0)
    pltpu.sync_copy(hr_hbm.at[pl.ds(r0, ZCHUNK), pl.ds(c * width, width)], gb1)
    pltpu.sync_copy(rcp_hbm.at[pl.ds(r0, ZCHUNK)], rbuf)

    def row(r, _):
      rcp = rbuf[r, :][0]
      for kk in range(width // 16):
        sl = pl.ds(16 * kk, 16)
        gb1[r, sl] = gb0[r, sl] * rcp + gb1[r, sl]
      return 0

    lax.fori_loop(0, ZCHUNK, row, 0)
    pltpu.sync_copy(gb1, out_hbm.at[pl.ds(r0, ZCHUNK), pl.ds(c * width, width)])


def _make_seg_sum(width: int, finish: bool):
  if finish:
    out_types = jax.ShapeDtypeStruct((ACC_ROWS, NC * width), jnp.float32)
  else:
    out_types = jax.ShapeDtypeStruct((NC, ACC_ROWS, width), jnp.float32)
  scratch = [
      pltpu.VMEM((IBLOCKS, EDGE_BLK), jnp.int32),
      pltpu.VMEM((IBLOCKS, EDGE_BLK), jnp.int32),
      pltpu.VMEM((EDGE_BLK, width), jnp.float32),
      pltpu.VMEM((EDGE_BLK, width), jnp.float32),
      pltpu.VMEM((EDGE_BLK, width), jnp.float32),
  ]
  if finish:
    scratch.append(pltpu.VMEM((ZCHUNK, CNT_W), jnp.float32))
  scratch.append(pltpu.VMEM_SHARED((ACC_ROWS, width), jnp.float32))
  scratch += [pltpu.SemaphoreType.DMA] * 6

  def body(*args):
    return _seg_body(width, finish, *args)

  return pl.kernel(
      body,
      out_type=out_types,
      mesh=_mesh,
      scratch_types=scratch,
      compiler_params=pltpu.CompilerParams(use_tc_tiling_on_sc=False),
  )


_seg_sum_l1 = _make_seg_sum(H1A, finish=False)
_seg_sum_l2 = _make_seg_sum(H2, finish=True)

ROW_BLK = 1000
GRID = N_NODES // ROW_BLK


def _dot(a, b):
  return lax.dot_general(a, b, (((1,), (0,)), ((), ())),
                         precision=lax.Precision.HIGHEST,
                         preferred_element_type=jnp.float32)


def _stage_a_body(x_ref, wl_ref, wr_ref, y_ref, xr_ref):
  xb = x_ref[...]
  y = _dot(xb, wl_ref[...])
  ones = jnp.ones((ROW_BLK, CNT_W), jnp.float32)
  y_ref[0] = jnp.concatenate([y[:, :H1], ones], axis=1)
  y_ref[1] = jnp.concatenate([y[:, H1:], ones], axis=1)
  xr_ref[...] = _dot(xb, wr_ref[...])


def _stage_a(x, w1l, w1r):
  return pl.pallas_call(
      _stage_a_body,
      grid=(GRID,),
      in_specs=[
          pl.BlockSpec((ROW_BLK, IN_DIM), lambda i: (i, 0)),
          pl.BlockSpec((IN_DIM, HID_DIM), lambda i: (0, 0)),
          pl.BlockSpec((IN_DIM, HID_DIM), lambda i: (0, 0)),
      ],
      out_specs=[
          pl.BlockSpec((NC, ROW_BLK, H1A), lambda i: (0, i, 0)),
          pl.BlockSpec((ROW_BLK, HID_DIM), lambda i: (i, 0)),
      ],
      out_shape=[
          jax.ShapeDtypeStruct((NC, N_NODES, H1A), jnp.float32),
          jax.ShapeDtypeStruct((N_NODES, HID_DIM), jnp.float32),
      ],
  )(x, w1l, w1r)


def _stage_c_body(p_ref, xr_ref, b1_ref, b2_ref, w2l_ref, w2r_ref,
                  y2_ref, hr_ref, rcp_ref):
  cnt = p_ref[0, :, H1:H1 + 1]
  rcp = 1.0 / jnp.maximum(cnt, 1.0)
  mean = jnp.concatenate([p_ref[0, :, :H1], p_ref[1, :, :H1]], axis=1) * rcp
  h = mean + xr_ref[...] + b1_ref[...]
  h = jnp.maximum(h, 0.0)
  y2 = _dot(h, w2l_ref[...])
  y2_ref[0] = y2[:, :H2]
  y2_ref[1] = y2[:, H2:]
  hr_ref[...] = _dot(h, w2r_ref[...]) + b2_ref[...]
  rcp_ref[...] = jnp.broadcast_to(rcp, (ROW_BLK, CNT_W))


def _stage_c(p, xr, b1, b2, w2l, w2r):
  return pl.pallas_call(
      _stage_c_body,
      grid=(GRID,),
      in_specs=[
          pl.BlockSpec((NC, ROW_BLK, H1A), lambda i: (0, i, 0)),
          pl.BlockSpec((ROW_BLK, HID_DIM), lambda i: (i, 0)),
          pl.BlockSpec((1, HID_DIM), lambda i: (0, 0)),
          pl.BlockSpec((1, OUT_DIM), lambda i: (0, 0)),
          pl.BlockSpec((HID_DIM, OUT_DIM), lambda i: (0, 0)),
          pl.BlockSpec((HID_DIM, OUT_DIM), lambda i: (0, 0)),
      ],
      out_specs=[
          pl.BlockSpec((NC, ROW_BLK, H2), lambda i: (0, i, 0)),
          pl.BlockSpec((ROW_BLK, OUT_DIM), lambda i: (i, 0)),
          pl.BlockSpec((ROW_BLK, CNT_W), lambda i: (i, 0)),
      ],
      out_shape=[
          jax.ShapeDtypeStruct((NC, N_NODES, H2), jnp.float32),
          jax.ShapeDtypeStruct((ACC_ROWS, OUT_DIM), jnp.float32),
          jax.ShapeDtypeStruct((ACC_ROWS, CNT_W), jnp.float32),
      ],
  )(p, xr, b1, b2, w2l, w2r)


@jax.jit
def kernel(x, edge_index, W1_l, W1_r, b1, W2_l, W2_r, b2):
  src = edge_index[0].astype(jnp.int32)
  dst = edge_index[1].astype(jnp.int32)
  pad = EDGES_PAD - N_EDGES
  src_p = jnp.concatenate([src, jnp.zeros((pad,), jnp.int32)])
  src3 = jnp.stack([src_p, src_p + N_NODES]).reshape(NC, NS, IBLOCKS, EDGE_BLK)
  dst3 = jnp.concatenate([dst, jnp.full((pad,), N_NODES, jnp.int32)])
  dst3 = dst3.reshape(NS, IBLOCKS, EDGE_BLK)

  z1 = jnp.zeros((ZCHUNK, H1A), jnp.float32)
  z2 = jnp.zeros((ZCHUNK, H2), jnp.float32)

  y1, xr = _stage_a(x, W1_l, W1_r)
  p1 = _seg_sum_l1(y1.reshape(NC * N_NODES, H1A), src3, dst3, z1)
  y2, hr2, rcp16 = _stage_c(p1, xr, b1.reshape(1, HID_DIM),
                            b2.reshape(1, OUT_DIM), W2_l, W2_r)
  out = _seg_sum_l2(y2.reshape(NC * N_NODES, H2), src3, dst3, z2, hr2, rcp16)
  return out[:N_NODES]


# speedup vs baseline: 1.2533x; 1.2533x over previous
import jax
import jax.numpy as jnp
from jax import lax
from jax.experimental import pallas as pl
from jax.experimental.pallas import tpu as pltpu
from jax.experimental.pallas import tpu_sc as plsc

N_NODES = 10000
IN_DIM = 256
HID_DIM = 128
OUT_DIM = 64
N_EDGES = 160000

NC = 2
NS = 16
EDGE_BLK = 128
NBUF = 2
EDGES_PER_TILE = -(-N_EDGES // NS)
IBLOCKS = -(-EDGES_PER_TILE // EDGE_BLK)
IBLOCKS = -(-IBLOCKS // NBUF) * NBUF
EDGES_PAD = NS * IBLOCKS * EDGE_BLK
NGROUPS = IBLOCKS // NBUF
ACC_ROWS = 10240
ROWS_PER_TILE = ACC_ROWS // NS
ZCHUNK = 128
CNT_W = 16
H1 = HID_DIM // 2
H2 = OUT_DIM // 2

_mesh = plsc.VectorSubcoreMesh(core_axis_name="c", subcore_axis_name="s",
                               num_cores=NC, num_subcores=NS)


def _seg_body(width, with_counts, finish, y_hbm, src_hbm, dst_hbm, zacc_hbm,
              *rest):
  if with_counts:
    (zcnt_hbm, ones_hbm, out_hbm, cnt_hbm, src_v, dst_v, gb0, gb1, acc,
     gsem0, gsem1, ssem0, ssem1, ones_v, cacc, csem) = rest
  elif finish:
    (hr_hbm, rcp_hbm, out_hbm, src_v, dst_v, gb0, gb1, rbuf, acc,
     gsem0, gsem1, ssem0, ssem1) = rest
  gbuf = (gb0, gb1)
  gsem = (gsem0, gsem1)
  ssem = (ssem0, ssem1)
  c = lax.axis_index("c")
  s = lax.axis_index("s")
  base = s * ROWS_PER_TILE

  pltpu.sync_copy(src_hbm.at[c, s], src_v)
  pltpu.sync_copy(dst_hbm.at[s], dst_v)
  for k in range(ROWS_PER_TILE // ZCHUNK):
    pltpu.sync_copy(zacc_hbm, acc.at[pl.ds(base + k * ZCHUNK, ZCHUNK)])
  if with_counts:
    pltpu.sync_copy(ones_hbm, ones_v)
    for k in range(ROWS_PER_TILE // ZCHUNK):
      pltpu.sync_copy(zcnt_hbm, cacc.at[pl.ds(base + k * ZCHUNK, ZCHUNK)])
  plsc.subcore_barrier()

  for b in range(NBUF):
    pltpu.async_copy(y_hbm.at[src_v.at[b]], gbuf[b], gsem[b])

  def group(g, _):
    for b in range(NBUF):
      j = g * NBUF + b
      pltpu.make_async_copy(y_hbm.at[src_v.at[j]], gbuf[b], gsem[b]).wait()
      pltpu.async_copy(gbuf[b], acc.at[dst_v.at[j]], ssem[b], add=True)
      if with_counts:
        @pl.when((j % NC) == c)
        def _():
          pltpu.async_copy(ones_v, cacc.at[dst_v.at[j]], csem, add=True)
    for b in range(NBUF):
      jn = (g + 1) * NBUF + b

      @pl.when(jn < IBLOCKS)
      def _():
        pltpu.make_async_copy(gbuf[b], acc.at[dst_v.at[jn]], ssem[b]).wait()
        pltpu.async_copy(y_hbm.at[src_v.at[jn]], gbuf[b], gsem[b])
    return 0

  lax.fori_loop(0, NGROUPS, group, 0)

  for b in range(NBUF):
    pltpu.make_async_copy(gbuf[b], acc.at[dst_v.at[0]], ssem[b]).wait()
  if with_counts:
    def drain(j, _):
      @pl.when((j % NC) == c)
      def _():
        pltpu.make_async_copy(ones_v, cacc.at[dst_v.at[0]], csem).wait()
      return 0
    lax.fori_loop(0, IBLOCKS, drain, 0)
  plsc.subcore_barrier()

  if not finish:
    pltpu.sync_copy(acc.at[pl.ds(base, ROWS_PER_TILE)],
                    out_hbm.at[c, pl.ds(base, ROWS_PER_TILE)])
    if with_counts:
      pltpu.sync_copy(cacc.at[pl.ds(base, ROWS_PER_TILE)],
                      cnt_hbm.at[c, pl.ds(base, ROWS_PER_TILE)])
    return

  for k in range(ROWS_PER_TILE // ZCHUNK):
    r0 = base + k * ZCHUNK
    pltpu.sync_copy(acc.at[pl.ds(r0, ZCHUNK)], gb0)
    pltpu.sync_copy(hr_hbm.at[pl.ds(r0, ZCHUNK), pl.ds(c * width, width)], gb1)
    pltpu.sync_copy(rcp_hbm.at[pl.ds(r0, ZCHUNK)], rbuf)

    def row(r, _):
      rcp = rbuf[r, :][0]
      for kk in range(width // 16):
        sl = pl.ds(16 * kk, 16)
        gb1[r, sl] = gb0[r, sl] * rcp + gb1[r, sl]
      return 0

    lax.fori_loop(0, ZCHUNK, row, 0)
    pltpu.sync_copy(gb1, out_hbm.at[pl.ds(r0, ZCHUNK), pl.ds(c * width, width)])


def _make_seg_sum(width: int, with_counts: bool, finish: bool):
  if finish:
    out_types = [jax.ShapeDtypeStruct((ACC_ROWS, NC * width), jnp.float32)]
  else:
    out_types = [jax.ShapeDtypeStruct((NC, ACC_ROWS, width), jnp.float32)]
  scratch = [
      pltpu.VMEM((IBLOCKS, EDGE_BLK), jnp.int32),
      pltpu.VMEM((IBLOCKS, EDGE_BLK), jnp.int32),
      pltpu.VMEM((EDGE_BLK, width), jnp.float32),
      pltpu.VMEM((EDGE_BLK, width), jnp.float32),
  ]
  if finish:
    scratch.append(pltpu.VMEM((ZCHUNK, CNT_W), jnp.float32))
  scratch.append(pltpu.VMEM_SHARED((ACC_ROWS, width), jnp.float32))
  scratch += [pltpu.SemaphoreType.DMA] * 4
  if with_counts:
    out_types.append(
        jax.ShapeDtypeStruct((NC, ACC_ROWS, CNT_W), jnp.float32))
    scratch += [
        pltpu.VMEM((EDGE_BLK, CNT_W), jnp.float32),
        pltpu.VMEM_SHARED((ACC_ROWS, CNT_W), jnp.float32),
        pltpu.SemaphoreType.DMA,
    ]

  def body(*args):
    return _seg_body(width, with_counts, finish, *args)

  return pl.kernel(
      body,
      out_type=out_types,
      mesh=_mesh,
      scratch_types=scratch,
      compiler_params=pltpu.CompilerParams(use_tc_tiling_on_sc=False),
  )


_seg_sum_l1 = _make_seg_sum(H1, with_counts=True, finish=False)
_seg_sum_l2 = _make_seg_sum(H2, with_counts=False, finish=True)

ROW_BLK = 1000
GRID = N_NODES // ROW_BLK


def _dot(a, b):
  return lax.dot_general(a, b, (((1,), (0,)), ((), ())),
                         precision=lax.Precision.HIGHEST,
                         preferred_element_type=jnp.float32)


def _stage_a_body(x_ref, wl_ref, wr_ref, y_ref, xr_ref):
  xb = x_ref[...]
  y = _dot(xb, wl_ref[...])
  y_ref[0] = y[:, :H1]
  y_ref[1] = y[:, H1:]
  xr_ref[...] = _dot(xb, wr_ref[...])


def _stage_a(x, w1l, w1r):
  return pl.pallas_call(
      _stage_a_body,
      grid=(GRID,),
      in_specs=[
          pl.BlockSpec((ROW_BLK, IN_DIM), lambda i: (i, 0)),
          pl.BlockSpec((IN_DIM, HID_DIM), lambda i: (0, 0)),
          pl.BlockSpec((IN_DIM, HID_DIM), lambda i: (0, 0)),
      ],
      out_specs=[
          pl.BlockSpec((NC, ROW_BLK, H1), lambda i: (0, i, 0)),
          pl.BlockSpec((ROW_BLK, HID_DIM), lambda i: (i, 0)),
      ],
      out_shape=[
          jax.ShapeDtypeStruct((NC, N_NODES, H1), jnp.float32),
          jax.ShapeDtypeStruct((N_NODES, HID_DIM), jnp.float32),
      ],
  )(x, w1l, w1r)


def _stage_c_body(p_ref, c_ref, xr_ref, b1_ref, b2_ref, w2l_ref, w2r_ref,
                  y2_ref, hr_ref, rcp_ref):
  cnt = c_ref[0, :, 0:1] + c_ref[1, :, 0:1]
  rcp = 1.0 / jnp.maximum(cnt, 1.0)
  mean = jnp.concatenate([p_ref[0], p_ref[1]], axis=1) * rcp
  h = mean + xr_ref[...] + b1_ref[...]
  h = jnp.maximum(h, 0.0)
  y2 = _dot(h, w2l_ref[...])
  y2_ref[0] = y2[:, :H2]
  y2_ref[1] = y2[:, H2:]
  hr_ref[...] = _dot(h, w2r_ref[...]) + b2_ref[...]
  rcp_ref[...] = jnp.broadcast_to(rcp, (ROW_BLK, CNT_W))


def _stage_c(p, cnt, xr, b1, b2, w2l, w2r):
  return pl.pallas_call(
      _stage_c_body,
      grid=(GRID,),
      in_specs=[
          pl.BlockSpec((NC, ROW_BLK, H1), lambda i: (0, i, 0)),
          pl.BlockSpec((NC, ROW_BLK, CNT_W), lambda i: (0, i, 0)),
          pl.BlockSpec((ROW_BLK, HID_DIM), lambda i: (i, 0)),
          pl.BlockSpec((1, HID_DIM), lambda i: (0, 0)),
          pl.BlockSpec((1, OUT_DIM), lambda i: (0, 0)),
          pl.BlockSpec((HID_DIM, OUT_DIM), lambda i: (0, 0)),
          pl.BlockSpec((HID_DIM, OUT_DIM), lambda i: (0, 0)),
      ],
      out_specs=[
          pl.BlockSpec((NC, ROW_BLK, H2), lambda i: (0, i, 0)),
          pl.BlockSpec((ROW_BLK, OUT_DIM), lambda i: (i, 0)),
          pl.BlockSpec((ROW_BLK, CNT_W), lambda i: (i, 0)),
      ],
      out_shape=[
          jax.ShapeDtypeStruct((NC, N_NODES, H2), jnp.float32),
          jax.ShapeDtypeStruct((ACC_ROWS, OUT_DIM), jnp.float32),
          jax.ShapeDtypeStruct((ACC_ROWS, CNT_W), jnp.float32),
      ],
  )(p, cnt, xr, b1, b2, w2l, w2r)


@jax.jit
def kernel(x, edge_index, W1_l, W1_r, b1, W2_l, W2_r, b2):
  src = edge_index[0].astype(jnp.int32)
  dst = edge_index[1].astype(jnp.int32)
  pad = EDGES_PAD - N_EDGES
  src_p = jnp.concatenate([src, jnp.zeros((pad,), jnp.int32)])
  src3 = jnp.stack([src_p, src_p + N_NODES]).reshape(NC, NS, IBLOCKS, EDGE_BLK)
  dst3 = jnp.concatenate([dst, jnp.full((pad,), N_NODES, jnp.int32)])
  dst3 = dst3.reshape(NS, IBLOCKS, EDGE_BLK)

  z1 = jnp.zeros((ZCHUNK, H1), jnp.float32)
  z2 = jnp.zeros((ZCHUNK, H2), jnp.float32)
  zc = jnp.zeros((ZCHUNK, CNT_W), jnp.float32)
  ones = jnp.ones((EDGE_BLK, CNT_W), jnp.float32)

  y1, xr = _stage_a(x, W1_l, W1_r)
  p1, c1 = _seg_sum_l1(y1.reshape(NC * N_NODES, H1), src3, dst3, z1, zc, ones)
  y2, hr2, rcp16 = _stage_c(p1, c1, xr, b1.reshape(1, HID_DIM),
                            b2.reshape(1, OUT_DIM), W2_l, W2_r)
  (out,) = _seg_sum_l2(y2.reshape(NC * N_NODES, H2), src3, dst3, z2,
                       hr2, rcp16)
  return out[:N_NODES]


# speedup vs baseline: 1.2939x; 1.0324x over previous
import jax
import jax.numpy as jnp
from jax import lax
from jax.experimental import pallas as pl
from jax.experimental.pallas import tpu as pltpu
from jax.experimental.pallas import tpu_sc as plsc

N_NODES = 10000
IN_DIM = 256
HID_DIM = 128
OUT_DIM = 64
N_EDGES = 160000

NC = 2
NS = 16
EDGE_BLK = 128
NBUF = 2
EDGES_PER_TILE = -(-N_EDGES // NS)
IBLOCKS = -(-EDGES_PER_TILE // EDGE_BLK)
IBLOCKS = -(-IBLOCKS // NBUF) * NBUF
EDGES_PAD = NS * IBLOCKS * EDGE_BLK
NGROUPS = IBLOCKS // NBUF
ACC_ROWS = 10240
ROWS_PER_TILE = ACC_ROWS // NS
ZCHUNK = 128
CNT_W = 16
H1 = HID_DIM // 2
H2 = OUT_DIM // 2

_mesh = plsc.VectorSubcoreMesh(core_axis_name="c", subcore_axis_name="s",
                               num_cores=NC, num_subcores=NS)


def _seg_body(width, with_counts, finish, bf, y_hbm, src_hbm, dst_hbm,
              zacc_hbm, *rest):
  nblk = IBLOCKS // bf
  ngroups = nblk // NBUF
  if with_counts:
    (zcnt_hbm, ones_hbm, out_hbm, cnt_hbm, src_v, dst_v, gb0, gb1, acc,
     gsem0, gsem1, ssem0, ssem1, ones_v, cacc, csem) = rest
  elif finish:
    (hr_hbm, rcp_hbm, out_hbm, src_v, dst_v, gb0, gb1, rbuf, acc,
     gsem0, gsem1, ssem0, ssem1) = rest
  gbuf = (gb0, gb1)
  gsem = (gsem0, gsem1)
  ssem = (ssem0, ssem1)
  c = lax.axis_index("c")
  s = lax.axis_index("s")
  base = s * ROWS_PER_TILE

  pltpu.sync_copy(src_hbm.at[c, s], src_v)
  pltpu.sync_copy(dst_hbm.at[s], dst_v)
  for k in range(ROWS_PER_TILE // ZCHUNK):
    pltpu.sync_copy(zacc_hbm, acc.at[pl.ds(base + k * ZCHUNK, ZCHUNK)])
  if with_counts:
    pltpu.sync_copy(ones_hbm, ones_v)
    for k in range(ROWS_PER_TILE // ZCHUNK):
      pltpu.sync_copy(zcnt_hbm, cacc.at[pl.ds(base + k * ZCHUNK, ZCHUNK)])
  plsc.subcore_barrier()

  def src_at(j):
    return src_v.at[j] if bf == 1 else src_v.at[pl.ds(j * bf, bf)]

  def dst_at(j):
    return dst_v.at[j] if bf == 1 else dst_v.at[pl.ds(j * bf, bf)]

  for b in range(NBUF):
    pltpu.async_copy(y_hbm.at[src_at(b)], gbuf[b], gsem[b])

  def group(g, _):
    for b in range(NBUF):
      j = g * NBUF + b
      pltpu.make_async_copy(y_hbm.at[src_at(j)], gbuf[b], gsem[b]).wait()
      pltpu.async_copy(gbuf[b], acc.at[dst_at(j)], ssem[b], add=True)
      if with_counts:
        @pl.when((j % NC) == c)
        def _():
          pltpu.async_copy(ones_v, cacc.at[dst_v.at[j]], csem, add=True)
    for b in range(NBUF):
      jn = (g + 1) * NBUF + b

      @pl.when(jn < nblk)
      def _():
        pltpu.make_async_copy(gbuf[b], acc.at[dst_at(jn)], ssem[b]).wait()
        pltpu.async_copy(y_hbm.at[src_at(jn)], gbuf[b], gsem[b])
    return 0

  lax.fori_loop(0, ngroups, group, 0)

  for b in range(NBUF):
    pltpu.make_async_copy(gbuf[b], acc.at[dst_at(0)], ssem[b]).wait()
  if with_counts:
    def drain(j, _):
      @pl.when((j % NC) == c)
      def _():
        pltpu.make_async_copy(ones_v, cacc.at[dst_v.at[0]], csem).wait()
      return 0
    lax.fori_loop(0, IBLOCKS, drain, 0)
  plsc.subcore_barrier()

  if not finish:
    pltpu.sync_copy(acc.at[pl.ds(base, ROWS_PER_TILE)],
                    out_hbm.at[c, pl.ds(base, ROWS_PER_TILE)])
    if with_counts:
      pltpu.sync_copy(cacc.at[pl.ds(base, ROWS_PER_TILE)],
                      cnt_hbm.at[c, pl.ds(base, ROWS_PER_TILE)])
    return

  for k in range(ROWS_PER_TILE // ZCHUNK):
    r0 = base + k * ZCHUNK
    pltpu.sync_copy(acc.at[pl.ds(r0, ZCHUNK)], gb0.at[pl.ds(0, ZCHUNK)])
    pltpu.sync_copy(hr_hbm.at[pl.ds(r0, ZCHUNK), pl.ds(c * width, width)],
                    gb1.at[pl.ds(0, ZCHUNK)])
    pltpu.sync_copy(rcp_hbm.at[pl.ds(r0, ZCHUNK)], rbuf)

    def row(r, _):
      rcp = rbuf[r, :][0]
      for kk in range(width // 16):
        sl = pl.ds(16 * kk, 16)
        gb1[r, sl] = gb0[r, sl] * rcp + gb1[r, sl]
      return 0

    lax.fori_loop(0, ZCHUNK, row, 0)
    pltpu.sync_copy(gb1.at[pl.ds(0, ZCHUNK)],
                    out_hbm.at[pl.ds(r0, ZCHUNK), pl.ds(c * width, width)])


def _make_seg_sum(width: int, with_counts: bool, finish: bool, bf: int = 1):
  if finish:
    out_types = [jax.ShapeDtypeStruct((ACC_ROWS, NC * width), jnp.float32)]
  else:
    out_types = [jax.ShapeDtypeStruct((NC, ACC_ROWS, width), jnp.float32)]
  scratch = [
      pltpu.VMEM((IBLOCKS, EDGE_BLK), jnp.int32),
      pltpu.VMEM((IBLOCKS, EDGE_BLK), jnp.int32),
      pltpu.VMEM((bf * EDGE_BLK, width), jnp.float32),
      pltpu.VMEM((bf * EDGE_BLK, width), jnp.float32),
  ]
  if finish:
    scratch.append(pltpu.VMEM((ZCHUNK, CNT_W), jnp.float32))
  scratch.append(pltpu.VMEM_SHARED((ACC_ROWS, width), jnp.float32))
  scratch += [pltpu.SemaphoreType.DMA] * 4
  if with_counts:
    out_types.append(
        jax.ShapeDtypeStruct((NC, ACC_ROWS, CNT_W), jnp.float32))
    scratch += [
        pltpu.VMEM((EDGE_BLK, CNT_W), jnp.float32),
        pltpu.VMEM_SHARED((ACC_ROWS, CNT_W), jnp.float32),
        pltpu.SemaphoreType.DMA,
    ]

  def body(*args):
    return _seg_body(width, with_counts, finish, bf, *args)

  return pl.kernel(
      body,
      out_type=out_types,
      mesh=_mesh,
      scratch_types=scratch,
      compiler_params=pltpu.CompilerParams(use_tc_tiling_on_sc=False),
  )


_seg_sum_l1 = _make_seg_sum(H1, with_counts=True, finish=False)
_seg_sum_l2 = _make_seg_sum(H2, with_counts=False, finish=True)

ROW_BLK = 1000
GRID = N_NODES // ROW_BLK


def _dot(a, b):
  return lax.dot_general(a, b, (((1,), (0,)), ((), ())),
                         precision=lax.Precision.DEFAULT,
                         preferred_element_type=jnp.float32)


def _stage_a_body(x_ref, wl_ref, wr_ref, y_ref, xr_ref):
  xb = x_ref[...]
  y = _dot(xb, wl_ref[...])
  y_ref[0] = y[:, :H1]
  y_ref[1] = y[:, H1:]
  xr_ref[...] = _dot(xb, wr_ref[...])


def _stage_a(x, w1l, w1r):
  return pl.pallas_call(
      _stage_a_body,
      grid=(GRID,),
      in_specs=[
          pl.BlockSpec((ROW_BLK, IN_DIM), lambda i: (i, 0)),
          pl.BlockSpec((IN_DIM, HID_DIM), lambda i: (0, 0)),
          pl.BlockSpec((IN_DIM, HID_DIM), lambda i: (0, 0)),
      ],
      out_specs=[
          pl.BlockSpec((NC, ROW_BLK, H1), lambda i: (0, i, 0)),
          pl.BlockSpec((ROW_BLK, HID_DIM), lambda i: (i, 0)),
      ],
      out_shape=[
          jax.ShapeDtypeStruct((NC, N_NODES, H1), jnp.float32),
          jax.ShapeDtypeStruct((N_NODES, HID_DIM), jnp.float32),
      ],
  )(x, w1l, w1r)


def _stage_c_body(p_ref, c_ref, xr_ref, b1_ref, b2_ref, w2l_ref, w2r_ref,
                  y2_ref, hr_ref, rcp_ref):
  cnt = c_ref[0, :, 0:1] + c_ref[1, :, 0:1]
  rcp = 1.0 / jnp.maximum(cnt, 1.0)
  mean = jnp.concatenate([p_ref[0], p_ref[1]], axis=1) * rcp
  h = mean + xr_ref[...] + b1_ref[...]
  h = jnp.maximum(h, 0.0)
  y2 = _dot(h, w2l_ref[...])
  y2_ref[0] = y2[:, :H2]
  y2_ref[1] = y2[:, H2:]
  hr_ref[...] = _dot(h, w2r_ref[...]) + b2_ref[...]
  rcp_ref[...] = jnp.broadcast_to(rcp, (ROW_BLK, CNT_W))


def _stage_c(p, cnt, xr, b1, b2, w2l, w2r):
  return pl.pallas_call(
      _stage_c_body,
      grid=(GRID,),
      in_specs=[
          pl.BlockSpec((NC, ROW_BLK, H1), lambda i: (0, i, 0)),
          pl.BlockSpec((NC, ROW_BLK, CNT_W), lambda i: (0, i, 0)),
          pl.BlockSpec((ROW_BLK, HID_DIM), lambda i: (i, 0)),
          pl.BlockSpec((1, HID_DIM), lambda i: (0, 0)),
          pl.BlockSpec((1, OUT_DIM), lambda i: (0, 0)),
          pl.BlockSpec((HID_DIM, OUT_DIM), lambda i: (0, 0)),
          pl.BlockSpec((HID_DIM, OUT_DIM), lambda i: (0, 0)),
      ],
      out_specs=[
          pl.BlockSpec((NC, ROW_BLK, H2), lambda i: (0, i, 0)),
          pl.BlockSpec((ROW_BLK, OUT_DIM), lambda i: (i, 0)),
          pl.BlockSpec((ROW_BLK, CNT_W), lambda i: (i, 0)),
      ],
      out_shape=[
          jax.ShapeDtypeStruct((NC, N_NODES, H2), jnp.float32),
          jax.ShapeDtypeStruct((ACC_ROWS, OUT_DIM), jnp.float32),
          jax.ShapeDtypeStruct((ACC_ROWS, CNT_W), jnp.float32),
      ],
  )(p, cnt, xr, b1, b2, w2l, w2r)


@jax.jit
def kernel(x, edge_index, W1_l, W1_r, b1, W2_l, W2_r, b2):
  src = edge_index[0].astype(jnp.int32)
  dst = edge_index[1].astype(jnp.int32)
  pad = EDGES_PAD - N_EDGES
  src_p = jnp.concatenate([src, jnp.zeros((pad,), jnp.int32)])
  src3 = jnp.stack([src_p, src_p + N_NODES]).reshape(NC, NS, IBLOCKS, EDGE_BLK)
  dst3 = jnp.concatenate([dst, jnp.full((pad,), N_NODES, jnp.int32)])
  dst3 = dst3.reshape(NS, IBLOCKS, EDGE_BLK)

  z1 = jnp.zeros((ZCHUNK, H1), jnp.float32)
  z2 = jnp.zeros((ZCHUNK, H2), jnp.float32)
  zc = jnp.zeros((ZCHUNK, CNT_W), jnp.float32)
  ones = jnp.ones((EDGE_BLK, CNT_W), jnp.float32)

  y1, xr = _stage_a(x, W1_l, W1_r)
  p1, c1 = _seg_sum_l1(y1.reshape(NC * N_NODES, H1), src3, dst3, z1, zc, ones)
  y2, hr2, rcp16 = _stage_c(p1, c1, xr, b1.reshape(1, HID_DIM),
                            b2.reshape(1, OUT_DIM), W2_l, W2_r)
  (out,) = _seg_sum_l2(y2.reshape(NC * N_NODES, H2), src3, dst3, z2,
                       hr2, rcp16)
  return out[:N_NODES]


# speedup vs baseline: 1.2986x; 1.0037x over previous
import jax
import jax.numpy as jnp
from jax import lax
from jax.experimental import pallas as pl
from jax.experimental.pallas import tpu as pltpu
from jax.experimental.pallas import tpu_sc as plsc

N_NODES = 10000
IN_DIM = 256
HID_DIM = 128
OUT_DIM = 64
N_EDGES = 160000

NC = 2
NS = 16
EDGE_BLK = 128
NBUF = 2
EDGES_PER_TILE = -(-N_EDGES // NS)
IBLOCKS = -(-EDGES_PER_TILE // EDGE_BLK)
IBLOCKS = -(-IBLOCKS // NBUF) * NBUF
EDGES_PAD = NS * IBLOCKS * EDGE_BLK
NGROUPS = IBLOCKS // NBUF
ACC_ROWS = 10240
ROWS_PER_TILE = ACC_ROWS // NS
ZCHUNK = 128
CNT_W = 16
H1 = HID_DIM // 2
H2 = OUT_DIM // 2

_mesh = plsc.VectorSubcoreMesh(core_axis_name="c", subcore_axis_name="s",
                               num_cores=NC, num_subcores=NS)


def _seg_body(width, with_counts, finish, bf, y_hbm, src_hbm, dst_hbm,
              zacc_hbm, *rest):
  nblk = IBLOCKS // bf
  ngroups = nblk // NBUF
  if with_counts:
    (zcnt_hbm, ones_hbm, out_hbm, cnt_hbm, src_v, dst_v, gb0, gb1, acc,
     gsem0, gsem1, ssem0, ssem1, ones_v, cacc, csem) = rest
  elif finish:
    (hr_hbm, rcp_hbm, out_hbm, src_v, dst_v, gb0, gb1, rbuf, acc,
     gsem0, gsem1, ssem0, ssem1) = rest
  gbuf = (gb0, gb1)
  gsem = (gsem0, gsem1)
  ssem = (ssem0, ssem1)
  c = lax.axis_index("c")
  s = lax.axis_index("s")
  base = s * ROWS_PER_TILE

  pltpu.sync_copy(src_hbm.at[c, s], src_v)
  pltpu.sync_copy(dst_hbm.at[s], dst_v)
  for k in range(ROWS_PER_TILE // ZCHUNK):
    pltpu.sync_copy(zacc_hbm, acc.at[pl.ds(base + k * ZCHUNK, ZCHUNK)])
  if with_counts:
    @pl.when(c == 0)
    def _():
      pltpu.sync_copy(ones_hbm, ones_v)
      for k in range(ROWS_PER_TILE // ZCHUNK):
        pltpu.sync_copy(zcnt_hbm, cacc.at[pl.ds(base + k * ZCHUNK, ZCHUNK)])
  plsc.subcore_barrier()

  def src_at(j):
    return src_v.at[j] if bf == 1 else src_v.at[pl.ds(j * bf, bf)]

  def dst_at(j):
    return dst_v.at[j] if bf == 1 else dst_v.at[pl.ds(j * bf, bf)]

  for b in range(NBUF):
    pltpu.async_copy(y_hbm.at[src_at(b)], gbuf[b], gsem[b])

  def group(g, _):
    for b in range(NBUF):
      j = g * NBUF + b
      pltpu.make_async_copy(y_hbm.at[src_at(j)], gbuf[b], gsem[b]).wait()
      pltpu.async_copy(gbuf[b], acc.at[dst_at(j)], ssem[b], add=True)
      if with_counts:
        @pl.when(c == 0)
        def _():
          pltpu.async_copy(ones_v, cacc.at[dst_v.at[j]], csem, add=True)
    for b in range(NBUF):
      jn = (g + 1) * NBUF + b

      @pl.when(jn < nblk)
      def _():
        pltpu.make_async_copy(gbuf[b], acc.at[dst_at(jn)], ssem[b]).wait()
        pltpu.async_copy(y_hbm.at[src_at(jn)], gbuf[b], gsem[b])
    return 0

  lax.fori_loop(0, ngroups, group, 0)

  for b in range(NBUF):
    pltpu.make_async_copy(gbuf[b], acc.at[dst_at(0)], ssem[b]).wait()
  if with_counts:
    @pl.when(c == 0)
    def _():
      def drain(j, _):
        pltpu.make_async_copy(ones_v, cacc.at[dst_v.at[0]], csem).wait()
        return 0
      lax.fori_loop(0, IBLOCKS, drain, 0)
  plsc.subcore_barrier()

  if not finish:
    pltpu.sync_copy(acc.at[pl.ds(base, ROWS_PER_TILE)],
                    out_hbm.at[c, pl.ds(base, ROWS_PER_TILE)])
    if with_counts:
      @pl.when(c == 0)
      def _():
        pltpu.sync_copy(cacc.at[pl.ds(base, ROWS_PER_TILE)],
                        cnt_hbm.at[pl.ds(base, ROWS_PER_TILE)])
    return

  for k in range(ROWS_PER_TILE // ZCHUNK):
    r0 = base + k * ZCHUNK
    pltpu.sync_copy(acc.at[pl.ds(r0, ZCHUNK)], gb0.at[pl.ds(0, ZCHUNK)])
    pltpu.sync_copy(hr_hbm.at[pl.ds(r0, ZCHUNK), pl.ds(c * width, width)],
                    gb1.at[pl.ds(0, ZCHUNK)])
    pltpu.sync_copy(rcp_hbm.at[pl.ds(r0, ZCHUNK)], rbuf)

    def row(r, _):
      rcp = rbuf[r, :][0]
      for kk in range(width // 16):
        sl = pl.ds(16 * kk, 16)
        gb1[r, sl] = gb0[r, sl] * rcp + gb1[r, sl]
      return 0

    lax.fori_loop(0, ZCHUNK, row, 0)
    pltpu.sync_copy(gb1.at[pl.ds(0, ZCHUNK)],
                    out_hbm.at[pl.ds(r0, ZCHUNK), pl.ds(c * width, width)])


def _make_seg_sum(width: int, with_counts: bool, finish: bool, bf: int = 1):
  if finish:
    out_types = [jax.ShapeDtypeStruct((ACC_ROWS, NC * width), jnp.float32)]
  else:
    out_types = [jax.ShapeDtypeStruct((NC, ACC_ROWS, width), jnp.float32)]
  scratch = [
      pltpu.VMEM((IBLOCKS, EDGE_BLK), jnp.int32),
      pltpu.VMEM((IBLOCKS, EDGE_BLK), jnp.int32),
      pltpu.VMEM((bf * EDGE_BLK, width), jnp.float32),
      pltpu.VMEM((bf * EDGE_BLK, width), jnp.float32),
  ]
  if finish:
    scratch.append(pltpu.VMEM((ZCHUNK, CNT_W), jnp.float32))
  scratch.append(pltpu.VMEM_SHARED((ACC_ROWS, width), jnp.float32))
  scratch += [pltpu.SemaphoreType.DMA] * 4
  if with_counts:
    out_types.append(jax.ShapeDtypeStruct((ACC_ROWS, CNT_W), jnp.float32))
    scratch += [
        pltpu.VMEM((EDGE_BLK, CNT_W), jnp.float32),
        pltpu.VMEM_SHARED((ACC_ROWS, CNT_W), jnp.float32),
        pltpu.SemaphoreType.DMA,
    ]

  def body(*args):
    return _seg_body(width, with_counts, finish, bf, *args)

  return pl.kernel(
      body,
      out_type=out_types,
      mesh=_mesh,
      scratch_types=scratch,
      compiler_params=pltpu.CompilerParams(use_tc_tiling_on_sc=False),
  )


_seg_sum_l1 = _make_seg_sum(H1, with_counts=True, finish=False)
_seg_sum_l2 = _make_seg_sum(H2, with_counts=False, finish=True)

ROW_BLK = 1000
GRID = N_NODES // ROW_BLK


def _dot(a, b):
  return lax.dot_general(a, b, (((1,), (0,)), ((), ())),
                         precision=lax.Precision.DEFAULT,
                         preferred_element_type=jnp.float32)


def _stage_a_body(x_ref, wl_ref, wr_ref, y_ref, xr_ref):
  xb = x_ref[...]
  y = _dot(xb, wl_ref[...])
  y_ref[0] = y[:, :H1]
  y_ref[1] = y[:, H1:]
  xr_ref[...] = _dot(xb, wr_ref[...])


def _stage_a(x, w1l, w1r):
  return pl.pallas_call(
      _stage_a_body,
      grid=(GRID,),
      in_specs=[
          pl.BlockSpec((ROW_BLK, IN_DIM), lambda i: (i, 0)),
          pl.BlockSpec((IN_DIM, HID_DIM), lambda i: (0, 0)),
          pl.BlockSpec((IN_DIM, HID_DIM), lambda i: (0, 0)),
      ],
      out_specs=[
          pl.BlockSpec((NC, ROW_BLK, H1), lambda i: (0, i, 0)),
          pl.BlockSpec((ROW_BLK, HID_DIM), lambda i: (i, 0)),
      ],
      out_shape=[
          jax.ShapeDtypeStruct((NC, N_NODES, H1), jnp.float32),
          jax.ShapeDtypeStruct((N_NODES, HID_DIM), jnp.float32),
      ],
  )(x, w1l, w1r)


def _stage_c_body(p_ref, c_ref, xr_ref, b1_ref, b2_ref, w2l_ref, w2r_ref,
                  y2_ref, hr_ref, rcp_ref):
  cnt = c_ref[:, 0:1]
  rcp = 1.0 / jnp.maximum(cnt, 1.0)
  mean = jnp.concatenate([p_ref[0], p_ref[1]], axis=1) * rcp
  h = mean + xr_ref[...] + b1_ref[...]
  h = jnp.maximum(h, 0.0)
  y2 = _dot(h, w2l_ref[...])
  y2_ref[0] = y2[:, :H2]
  y2_ref[1] = y2[:, H2:]
  hr_ref[...] = _dot(h, w2r_ref[...]) + b2_ref[...]
  rcp_ref[...] = jnp.broadcast_to(rcp, (ROW_BLK, CNT_W))


def _stage_c(p, cnt, xr, b1, b2, w2l, w2r):
  return pl.pallas_call(
      _stage_c_body,
      grid=(GRID,),
      in_specs=[
          pl.BlockSpec((NC, ROW_BLK, H1), lambda i: (0, i, 0)),
          pl.BlockSpec((ROW_BLK, CNT_W), lambda i: (i, 0)),
          pl.BlockSpec((ROW_BLK, HID_DIM), lambda i: (i, 0)),
          pl.BlockSpec((1, HID_DIM), lambda i: (0, 0)),
          pl.BlockSpec((1, OUT_DIM), lambda i: (0, 0)),
          pl.BlockSpec((HID_DIM, OUT_DIM), lambda i: (0, 0)),
          pl.BlockSpec((HID_DIM, OUT_DIM), lambda i: (0, 0)),
      ],
      out_specs=[
          pl.BlockSpec((NC, ROW_BLK, H2), lambda i: (0, i, 0)),
          pl.BlockSpec((ROW_BLK, OUT_DIM), lambda i: (i, 0)),
          pl.BlockSpec((ROW_BLK, CNT_W), lambda i: (i, 0)),
      ],
      out_shape=[
          jax.ShapeDtypeStruct((NC, N_NODES, H2), jnp.float32),
          jax.ShapeDtypeStruct((ACC_ROWS, OUT_DIM), jnp.float32),
          jax.ShapeDtypeStruct((ACC_ROWS, CNT_W), jnp.float32),
      ],
  )(p, cnt, xr, b1, b2, w2l, w2r)


@jax.jit
def kernel(x, edge_index, W1_l, W1_r, b1, W2_l, W2_r, b2):
  src = edge_index[0].astype(jnp.int32)
  dst = edge_index[1].astype(jnp.int32)
  pad = EDGES_PAD - N_EDGES
  src_p = jnp.concatenate([src, jnp.zeros((pad,), jnp.int32)])
  src3 = jnp.stack([src_p, src_p + N_NODES]).reshape(NC, NS, IBLOCKS, EDGE_BLK)
  dst3 = jnp.concatenate([dst, jnp.full((pad,), N_NODES, jnp.int32)])
  dst3 = dst3.reshape(NS, IBLOCKS, EDGE_BLK)

  z1 = jnp.zeros((ZCHUNK, H1), jnp.float32)
  z2 = jnp.zeros((ZCHUNK, H2), jnp.float32)
  zc = jnp.zeros((ZCHUNK, CNT_W), jnp.float32)
  ones = jnp.ones((EDGE_BLK, CNT_W), jnp.float32)

  y1, xr = _stage_a(x, W1_l, W1_r)
  p1, c1 = _seg_sum_l1(y1.reshape(NC * N_NODES, H1), src3, dst3, z1, zc, ones)
  y2, hr2, rcp16 = _stage_c(p1, c1, xr, b1.reshape(1, HID_DIM),
                            b2.reshape(1, OUT_DIM), W2_l, W2_r)
  (out,) = _seg_sum_l2(y2.reshape(NC * N_NODES, H2), src3, dst3, z2,
                       hr2, rcp16)
  return out[:N_NODES]
